# Initial kernel scaffold; baseline (speedup 1.0000x reference)
import functools

import jax
import jax.numpy as jnp
from jax import lax
from jax.experimental import pallas as pl
from jax.experimental.pallas import tpu as pltpu
from jax.experimental.pallas import tpu_sc as plsc

_NS = 16
_K = 128


def _sc_seg_gather(x0, x1, srcm, dstm, idxm, zrow, zcnt, oneb,
                   n_chunks, rows_pt, b_pt, np_rows, Dh, B):
    mesh = plsc.VectorSubcoreMesh(core_axis_name="c", subcore_axis_name="s")
    nq = b_pt // _K

    @functools.partial(
        pl.kernel, mesh=mesh,
        out_type=(
            jax.ShapeDtypeStruct((B, Dh), jnp.float32),
            jax.ShapeDtypeStruct((B, Dh), jnp.float32),
            jax.ShapeDtypeStruct((B, Dh), jnp.float32),
            jax.ShapeDtypeStruct((B, Dh), jnp.float32),
            jax.ShapeDtypeStruct((B, 16), jnp.float32),
        ),
        scratch_types=[
            pltpu.VMEM_SHARED((np_rows, Dh), jnp.float32),
            pltpu.VMEM_SHARED((np_rows, 16), jnp.float32),
            pltpu.VMEM((n_chunks, _K), jnp.int32),
            pltpu.VMEM((n_chunks, _K), jnp.int32),
            pltpu.VMEM((_K, Dh), jnp.float32),
            pltpu.VMEM((_K, Dh), jnp.float32),
            pltpu.VMEM((_K, 16), jnp.float32),
            pltpu.VMEM((max(b_pt // _K, 1), _K), jnp.int32),
            pltpu.VMEM((_K, 16), jnp.float32),
            pltpu.SemaphoreType.DMA,
            pltpu.SemaphoreType.DMA,
            pltpu.SemaphoreType.DMA,
        ],
    )
    def k(x0_h, x1_h, srcm_h, dstm_h, idxm_h, zrow_h, zcnt_h, oneb_h,
          a0_h, a1_h, xi0_h, xi1_h, cnt_h,
          aggr_sp, cnt_sp, src_t, dst_t, rows_a, rows_b, ones_v, idx_t,
          crows, sem_a, sem_b, sem_c):
        c = lax.axis_index("c")
        s = lax.axis_index("s")

        def run(x_h, a_h, xi_h, do_cnt):
            pltpu.sync_copy(zrow_h, aggr_sp.at[pl.ds(s * rows_pt, rows_pt)])
            if do_cnt:
                pltpu.sync_copy(zcnt_h, cnt_sp.at[pl.ds(s * rows_pt, rows_pt)])
                pltpu.sync_copy(oneb_h, ones_v)
            pltpu.sync_copy(srcm_h.at[s], src_t)
            pltpu.sync_copy(dstm_h.at[s], dst_t)
            plsc.subcore_barrier()

            def gstart(j, buf, sem):
                pltpu.make_async_copy(x_h.at[src_t.at[j]], buf, sem).start()

            def gwait(buf, sem):
                pltpu.make_async_copy(x_h.at[pl.ds(0, _K)], buf, sem).wait()

            gstart(0, rows_a, sem_a)
            gstart(1, rows_b, sem_b)

            def step(j, buf, sem):
                gwait(buf, sem)

                @pl.when(j + 2 < n_chunks)
                def _():
                    gstart(j + 2, buf, sem)

                pltpu.sync_copy(buf, aggr_sp.at[dst_t.at[j]], add=True)
                if do_cnt:
                    pltpu.sync_copy(ones_v, cnt_sp.at[dst_t.at[j]], add=True)

            def outer(jo, carry):
                step(2 * jo, rows_a, sem_a)
                step(2 * jo + 1, rows_b, sem_b)
                return carry

            lax.fori_loop(0, n_chunks // 2, outer, 0)
            plsc.subcore_barrier()

            pltpu.sync_copy(idxm_h.at[s], idx_t)
            base = s * b_pt
            for q in range(nq):
                idxq = idx_t.at[q]
                pltpu.async_copy(aggr_sp.at[idxq], rows_a, sem_a).wait()
                pltpu.sync_copy(rows_a, a_h.at[pl.ds(base + q * _K, _K)])
                pltpu.async_copy(x_h.at[idxq], rows_b, sem_b).wait()
                pltpu.sync_copy(rows_b, xi_h.at[pl.ds(base + q * _K, _K)])
                if do_cnt:
                    pltpu.async_copy(cnt_sp.at[idxq], crows, sem_c).wait()
                    pltpu.sync_copy(crows, cnt_h.at[pl.ds(base + q * _K, _K)])

        @pl.when(c == 0)
        def _():
            run(x0_h, a0_h, xi0_h, True)

        @pl.when(c == 1)
        def _():
            run(x1_h, a1_h, xi1_h, False)

    return k(x0, x1, srcm, dstm, idxm, zrow, zcnt, oneb)


def _ln(h, g, b):
    m = jnp.mean(h, axis=-1, keepdims=True)
    v = jnp.mean((h - m) ** 2, axis=-1, keepdims=True)
    return (h - m) * lax.rsqrt(v + 1e-5) * g + b


def _relu(h):
    return jnp.maximum(h, 0.0)


def _dot(a, w):
    return jnp.dot(a, w, preferred_element_type=jnp.float32)


def _tc_towers_body(H, a0, a1, cnt, xi0, xi1, qe,
                    Wl, bl, Wr, gg, bg, Wtf, btf, g1, b1, Wtf2, btf2, g2, b2,
                    Wq, bq, gq, bqv, Wf1, bf1, gf, bf, Wf2, bf2, Wf3, bf3,
                    out):
    a = jnp.concatenate([a0[...], a1[...]], axis=1)
    xi = jnp.concatenate([xi0[...], xi1[...]], axis=1)
    c0 = cnt[...][:, 0:1]
    a = a / jnp.maximum(c0, 1.0)

    h = _dot(a, Wl[...]) + bl[...] + _dot(xi, Wr[...])
    s = _relu(_ln(h, gg[...], bg[...]))
    t = _relu(_ln(_dot(xi, Wtf[...]) + btf[...], g1[...], b1[...]))
    t = _relu(_ln(_dot(t, Wtf2[...]) + btf2[...], g2[...], b2[...]))
    q = _relu(_ln(_dot(qe[...], Wq[...]) + bq[...], gq[...], bqv[...]))

    z = (_dot(s, Wf1[pl.ds(0, H), :]) + _dot(t, Wf1[pl.ds(H, H), :])
         + _dot(q, Wf1[pl.ds(2 * H, H), :]) + bf1[...])
    z = _relu(_ln(z, gf[...], bf[...]))
    y = _relu(_dot(z, Wf2[pl.ds(0, H), :]) + _dot(q, Wf2[pl.ds(H, H), :])
              + bf2[...])
    o = _dot(y, Wf3[...]) + bf3[...]
    out[...] = jax.nn.sigmoid(o)


def _tc_towers(a0, a1, cnt16, xi0, xi1, qe, Wl, bl, Wr, gg, bg,
               Wtf, btf, g1, b1, Wtf2, btf2, g2, b2, Wq, bq, gq, bqv,
               Wf1, bf1, gf, bf, Wf2, bf2, Wf3, bf3):
    B, Dh = a0.shape
    H = Wl.shape[1]
    OUT = Wf3.shape[1]
    R = 1024

    def row(shape):
        nd = len(shape)
        return pl.BlockSpec((R,) + tuple(shape[1:]),
                            lambda i, _nd=nd: (i,) + (0,) * (_nd - 1))

    def full(shape):
        nd = len(shape)
        return pl.BlockSpec(tuple(shape), lambda i, _nd=nd: (0,) * _nd)

    args = (a0, a1, cnt16, xi0, xi1, qe, Wl, bl, Wr, gg, bg,
            Wtf, btf, g1, b1, Wtf2, btf2, g2, b2, Wq, bq, gq, bqv,
            Wf1, bf1, gf, bf, Wf2, bf2, Wf3, bf3)
    in_specs = [row(a.shape) for a in args[:6]] + [
        full(w.shape) for w in args[6:]]
    return pl.pallas_call(
        functools.partial(_tc_towers_body, H),
        grid=(B // R,),
        in_specs=in_specs,
        out_specs=pl.BlockSpec((R, OUT), lambda i: (i, 0)),
        out_shape=jax.ShapeDtypeStruct((B, OUT), jnp.float32),
    )(*args)


def kernel(x, question_embedding, W_sage_l, b_sage_l, W_sage_r, gamma_g, beta_g,
           W_tf, b_tf, gamma_t1, beta_t1, W_tf2, b_tf2, gamma_t2, beta_t2,
           W_q, b_q, gamma_q, beta_q, W_f1, b_f1, gamma_f, beta_f,
           W_f2, b_f2, W_f3, b_f3, edge_index, idx):
    N, D = x.shape
    E = edge_index.shape[1]
    B = idx.shape[0]
    Dh = D // 2

    x0 = x[:, :Dh]
    x1 = x[:, Dh:]

    n_chunks = -(-E // (_NS * _K))
    epad = _NS * n_chunks * _K
    pad = epad - E
    src = edge_index[0]
    dst = edge_index[1]
    ndum = 128
    np_rows = -(-(N + ndum) // (_NS * 8)) * (_NS * 8)
    rows_pt = np_rows // _NS
    if pad:
        pidx = jnp.arange(pad, dtype=jnp.int32)
        src = jnp.concatenate([src, (pidx * 97) % N])
        dst = jnp.concatenate([dst, N + (pidx % ndum)])
    srcm = src.reshape(_NS, n_chunks, _K)
    dstm = dst.reshape(_NS, n_chunks, _K)

    b_pt = B // _NS
    idxm = idx.reshape(_NS, b_pt // _K, _K)

    zrow = jnp.zeros((rows_pt, Dh), jnp.float32)
    zcnt = jnp.zeros((rows_pt, 16), jnp.float32)
    oneb = jnp.ones((_K, 16), jnp.float32)

    a0, a1, xi0, xi1, cnt16 = _sc_seg_gather(
        x0, x1, srcm, dstm, idxm, zrow, zcnt, oneb,
        n_chunks, rows_pt, b_pt, np_rows, Dh, B)

    def r2(v):
        return v.reshape(1, -1)

    return _tc_towers(
        a0, a1, cnt16, xi0, xi1, question_embedding,
        W_sage_l, r2(b_sage_l), W_sage_r, r2(gamma_g), r2(beta_g),
        W_tf, r2(b_tf), r2(gamma_t1), r2(beta_t1),
        W_tf2, r2(b_tf2), r2(gamma_t2), r2(beta_t2),
        W_q, r2(b_q), r2(gamma_q), r2(beta_q),
        W_f1, r2(b_f1), r2(gamma_f), r2(beta_f),
        W_f2, r2(b_f2), W_f3, r2(b_f3))


# speedup vs baseline: 6.8922x; 6.8922x over previous
import functools

import jax
import jax.numpy as jnp
from jax import lax
from jax.experimental import pallas as pl
from jax.experimental.pallas import tpu as pltpu
from jax.experimental.pallas import tpu_sc as plsc

_NS = 16
_K = 128


def _sc_seg_gather(x0, x1, srcm, dstm, idxm, zrow, zcnt, oneb,
                   n_chunks, rows_pt, b_pt, np_rows, Dh, B):
    mesh = plsc.VectorSubcoreMesh(core_axis_name="c", subcore_axis_name="s")
    nq = b_pt // _K

    @functools.partial(
        pl.kernel, mesh=mesh,
        out_type=(
            jax.ShapeDtypeStruct((B, Dh), jnp.float32),
            jax.ShapeDtypeStruct((B, Dh), jnp.float32),
            jax.ShapeDtypeStruct((B, Dh), jnp.float32),
            jax.ShapeDtypeStruct((B, Dh), jnp.float32),
            jax.ShapeDtypeStruct((B,), jnp.float32),
        ),
        scratch_types=[
            pltpu.VMEM_SHARED((np_rows, Dh), jnp.float32),
            pltpu.VMEM_SHARED((np_rows,), jnp.float32),
            pltpu.VMEM((2, _K), jnp.int32),
            pltpu.VMEM((2, _K), jnp.int32),
            pltpu.VMEM((_K, Dh), jnp.float32),
            pltpu.VMEM((_K, Dh), jnp.float32),
            pltpu.VMEM((_K,), jnp.float32),
            pltpu.VMEM((max(b_pt // _K, 1), _K), jnp.int32),
            pltpu.VMEM((_K,), jnp.float32),
            pltpu.SemaphoreType.DMA,
            pltpu.SemaphoreType.DMA,
            pltpu.SemaphoreType.DMA,
        ],
    )
    def k(x0_h, x1_h, srcm_h, dstm_h, idxm_h, zrow_h, zcnt_h, oneb_h,
          a0_h, a1_h, xi0_h, xi1_h, cnt_h,
          aggr_sp, cnt_sp, src_t, dst_t, rows_a, rows_b, ones_v, idx_t,
          crows, sem_a, sem_b, sem_c):
        c = lax.axis_index("c")
        s = lax.axis_index("s")

        def run(x_h, a_h, xi_h, do_cnt):
            pltpu.sync_copy(zrow_h, aggr_sp.at[pl.ds(s * rows_pt, rows_pt)])
            if do_cnt:
                pltpu.sync_copy(zcnt_h, cnt_sp.at[pl.ds(s * rows_pt, rows_pt)])
                pltpu.sync_copy(oneb_h, ones_v)
            plsc.subcore_barrier()

            def stage(j, b):
                pltpu.sync_copy(srcm_h.at[s, j], src_t.at[b])
                pltpu.sync_copy(dstm_h.at[s, j], dst_t.at[b])

            def gstart(b, buf, sem):
                pltpu.make_async_copy(x_h.at[src_t.at[b]], buf, sem).start()

            def gwait(buf, sem):
                pltpu.make_async_copy(x_h.at[pl.ds(0, _K)], buf, sem).wait()

            stage(0, 0)
            gstart(0, rows_a, sem_a)
            stage(1, 1)
            gstart(1, rows_b, sem_b)

            def step(j, b, buf, sem):
                gwait(buf, sem)
                pltpu.sync_copy(buf, aggr_sp.at[dst_t.at[b]], add=True)
                if do_cnt:
                    pltpu.sync_copy(ones_v, cnt_sp.at[dst_t.at[b]], add=True)

                @pl.when(j + 2 < n_chunks)
                def _():
                    stage(j + 2, b)
                    gstart(b, buf, sem)

            def outer(jo, carry):
                step(2 * jo, 0, rows_a, sem_a)
                step(2 * jo + 1, 1, rows_b, sem_b)
                return carry

            lax.fori_loop(0, n_chunks // 2, outer, 0)
            plsc.subcore_barrier()

            pltpu.sync_copy(idxm_h.at[s], idx_t)
            base = s * b_pt
            for q in range(nq):
                idxq = idx_t.at[q]
                pltpu.async_copy(aggr_sp.at[idxq], rows_a, sem_a).wait()
                pltpu.sync_copy(rows_a, a_h.at[pl.ds(base + q * _K, _K)])
                pltpu.async_copy(x_h.at[idxq], rows_b, sem_b).wait()
                pltpu.sync_copy(rows_b, xi_h.at[pl.ds(base + q * _K, _K)])
                if do_cnt:
                    pltpu.async_copy(cnt_sp.at[idxq], crows, sem_c).wait()
                    pltpu.sync_copy(crows, cnt_h.at[pl.ds(base + q * _K, _K)])

        @pl.when(c == 0)
        def _():
            run(x0_h, a0_h, xi0_h, True)

        @pl.when(c == 1)
        def _():
            run(x1_h, a1_h, xi1_h, False)

    return k(x0, x1, srcm, dstm, idxm, zrow, zcnt, oneb)


def _ln(h, g, b):
    m = jnp.mean(h, axis=-1, keepdims=True)
    v = jnp.mean((h - m) ** 2, axis=-1, keepdims=True)
    return (h - m) * lax.rsqrt(v + 1e-5) * g + b


def _relu(h):
    return jnp.maximum(h, 0.0)


def _dot(a, w):
    return jnp.dot(a, w, preferred_element_type=jnp.float32)


def _tc_towers_body(H, a0, a1, cnt, xi0, xi1, qe,
                    Wl, bl, Wr, gg, bg, Wtf, btf, g1, b1, Wtf2, btf2, g2, b2,
                    Wq, bq, gq, bqv, Wf1, bf1, gf, bf, Wf2, bf2, Wf3, bf3,
                    out):
    a = jnp.concatenate([a0[...], a1[...]], axis=1)
    xi = jnp.concatenate([xi0[...], xi1[...]], axis=1)
    a = a / jnp.maximum(cnt[...], 1.0)

    h = _dot(a, Wl[...]) + bl[...] + _dot(xi, Wr[...])
    s = _relu(_ln(h, gg[...], bg[...]))
    t = _relu(_ln(_dot(xi, Wtf[...]) + btf[...], g1[...], b1[...]))
    t = _relu(_ln(_dot(t, Wtf2[...]) + btf2[...], g2[...], b2[...]))
    q = _relu(_ln(_dot(qe[...], Wq[...]) + bq[...], gq[...], bqv[...]))

    z = (_dot(s, Wf1[pl.ds(0, H), :]) + _dot(t, Wf1[pl.ds(H, H), :])
         + _dot(q, Wf1[pl.ds(2 * H, H), :]) + bf1[...])
    z = _relu(_ln(z, gf[...], bf[...]))
    y = _relu(_dot(z, Wf2[pl.ds(0, H), :]) + _dot(q, Wf2[pl.ds(H, H), :])
              + bf2[...])
    o = _dot(y, Wf3[...]) + bf3[...]
    out[...] = jax.nn.sigmoid(o)


def _tc_towers(a0, a1, cnt16, xi0, xi1, qe, Wl, bl, Wr, gg, bg,
               Wtf, btf, g1, b1, Wtf2, btf2, g2, b2, Wq, bq, gq, bqv,
               Wf1, bf1, gf, bf, Wf2, bf2, Wf3, bf3):
    B, Dh = a0.shape
    H = Wl.shape[1]
    OUT = Wf3.shape[1]
    R = 1024

    def row(shape):
        nd = len(shape)
        return pl.BlockSpec((R,) + tuple(shape[1:]),
                            lambda i, _nd=nd: (i,) + (0,) * (_nd - 1))

    def full(shape):
        nd = len(shape)
        return pl.BlockSpec(tuple(shape), lambda i, _nd=nd: (0,) * _nd)

    args = (a0, a1, cnt16, xi0, xi1, qe, Wl, bl, Wr, gg, bg,
            Wtf, btf, g1, b1, Wtf2, btf2, g2, b2, Wq, bq, gq, bqv,
            Wf1, bf1, gf, bf, Wf2, bf2, Wf3, bf3)
    in_specs = [row(a.shape) for a in args[:6]] + [
        full(w.shape) for w in args[6:]]
    return pl.pallas_call(
        functools.partial(_tc_towers_body, H),
        grid=(B // R,),
        in_specs=in_specs,
        out_specs=pl.BlockSpec((R, OUT), lambda i: (i, 0)),
        out_shape=jax.ShapeDtypeStruct((B, OUT), jnp.float32),
    )(*args)


def kernel(x, question_embedding, W_sage_l, b_sage_l, W_sage_r, gamma_g, beta_g,
           W_tf, b_tf, gamma_t1, beta_t1, W_tf2, b_tf2, gamma_t2, beta_t2,
           W_q, b_q, gamma_q, beta_q, W_f1, b_f1, gamma_f, beta_f,
           W_f2, b_f2, W_f3, b_f3, edge_index, idx):
    N, D = x.shape
    E = edge_index.shape[1]
    B = idx.shape[0]
    Dh = D // 2

    x0 = x[:, :Dh]
    x1 = x[:, Dh:]

    n_chunks = 2 * -(-E // (_NS * _K * 2))
    epad = _NS * n_chunks * _K
    pad = epad - E
    src = edge_index[0]
    dst = edge_index[1]
    ndum = 128
    np_rows = -(-(N + ndum) // (_NS * 8)) * (_NS * 8)
    rows_pt = np_rows // _NS
    if pad:
        pidx = jnp.arange(pad, dtype=jnp.int32)
        src = jnp.concatenate([src, (pidx * 97) % N])
        dst = jnp.concatenate([dst, N + (pidx % ndum)])
    srcm = src.reshape(_NS, n_chunks, _K)
    dstm = dst.reshape(_NS, n_chunks, _K)

    b_pt = B // _NS
    idxm = idx.reshape(_NS, b_pt // _K, _K)

    zrow = jnp.zeros((rows_pt, Dh), jnp.float32)
    zcnt = jnp.zeros((rows_pt,), jnp.float32)
    oneb = jnp.ones((_K,), jnp.float32)

    a0, a1, xi0, xi1, cnt = _sc_seg_gather(
        x0, x1, srcm, dstm, idxm, zrow, zcnt, oneb,
        n_chunks, rows_pt, b_pt, np_rows, Dh, B)

    def r2(v):
        return v.reshape(1, -1)

    return _tc_towers(
        a0, a1, cnt.reshape(B, 1), xi0, xi1, question_embedding,
        W_sage_l, r2(b_sage_l), W_sage_r, r2(gamma_g), r2(beta_g),
        W_tf, r2(b_tf), r2(gamma_t1), r2(beta_t1),
        W_tf2, r2(b_tf2), r2(gamma_t2), r2(beta_t2),
        W_q, r2(b_q), r2(gamma_q), r2(beta_q),
        W_f1, r2(b_f1), r2(gamma_f), r2(beta_f),
        W_f2, r2(b_f2), W_f3, r2(b_f3))
